# baseline (speedup 1.0000x reference)
import jax
import jax.numpy as jnp
from jax import lax
from jax.experimental import pallas as pl
from jax.experimental.pallas import tpu as pltpu
from jax.experimental.pallas import tpu_sc as plsc

_L = 20
_BATCH = 16384
_NC, _NS, _LANES = 2, 16, 16
_NW = _NC * _NS
_BPW = _BATCH // _NW
_NV = _BPW // _LANES
_GCH = 128
_NG = _BPW // _GCH

_LN2 = 0.6931471805599453
_SQRT2 = 1.4142135623730951


def _log16(a):
    bits = lax.bitcast_convert_type(a, jnp.int32)
    e = (bits >> 23) - 127
    m = lax.bitcast_convert_type((bits & 0x007FFFFF) | 0x3F800000, jnp.float32)
    big = m > _SQRT2
    m = jnp.where(big, m * 0.5, m)
    e = jnp.where(big, e + 1, e).astype(jnp.float32)
    s = (m - 1.0) / (m + 1.0)
    s2 = s * s
    p = jnp.float32(2.0 / 9.0)
    p = p * s2 + jnp.float32(2.0 / 7.0)
    p = p * s2 + jnp.float32(2.0 / 5.0)
    p = p * s2 + jnp.float32(2.0 / 3.0)
    p = p * s2 + jnp.float32(2.0)
    return e * jnp.float32(_LN2) + s * p


def _ewf_body(xt_hbm, aux_hbm, out_hbm, xv, idxv, av, outv, sem, gsem):
    wid = lax.axis_index("s") * _NC + lax.axis_index("c")
    base = pl.multiple_of(wid * _BPW, _BPW)

    cps = [
        pltpu.async_copy(
            xt_hbm.at[pl.ds(i * _BATCH + base, _BPW)],
            xv.at[i],
            sem,
        )
        for i in range(_L)
    ]
    for cp in cps:
        cp.wait()

    for v in range(_NV):
        acc = jnp.full((_LANES,), (1 << _L) - 1, jnp.int32)
        for i in range(_L):
            z = xv[i, pl.ds(v * _LANES, _LANES)]
            acc = acc + (z << (_L - 1 - i))
        g = v // (_GCH // _LANES)
        idxv[g, pl.ds((v % (_GCH // _LANES)) * _LANES, _LANES)] = acc >> 1

    gcps = [
        pltpu.async_copy(aux_hbm.at[idxv.at[g]], av.at[g], gsem)
        for g in range(_NG)
    ]
    for cp in gcps:
        cp.wait()

    for v in range(_NV):
        g = v // (_GCH // _LANES)
        sl = pl.ds((v % (_GCH // _LANES)) * _LANES, _LANES)
        outv[pl.ds(v * _LANES, _LANES)] = _log16(av[g, sl])

    pltpu.sync_copy(outv, out_hbm.at[pl.ds(base, _BPW)])


@jax.jit
def _ewf_sc(xt_flat, aux):
    mesh = plsc.VectorSubcoreMesh(core_axis_name="c", subcore_axis_name="s")
    return pl.kernel(
        _ewf_body,
        out_type=jax.ShapeDtypeStruct((_BATCH,), jnp.float32),
        mesh=mesh,
        scratch_types=[
            pltpu.VMEM((_L, _BPW), jnp.int32),
            pltpu.VMEM((_NG, _GCH), jnp.int32),
            pltpu.VMEM((_NG, _GCH), jnp.float32),
            pltpu.VMEM((_BPW,), jnp.float32),
            pltpu.SemaphoreType.DMA,
            pltpu.SemaphoreType.DMA,
        ],
    )(xt_flat, aux)


def kernel(x, aux):
    xt = jnp.transpose(x).reshape(-1)
    return _ewf_sc(xt, aux)


# speedup vs baseline: 1.0411x; 1.0411x over previous
import jax
import jax.numpy as jnp
from jax import lax
from jax.experimental import pallas as pl
from jax.experimental.pallas import tpu as pltpu
from jax.experimental.pallas import tpu_sc as plsc

_L = 20
_BATCH = 16384
_NC, _NS, _LANES = 2, 16, 16
_NW = _NC * _NS
_BPW = _BATCH // _NW
_GCH = 128
_NG = _BPW // _GCH
_VPG = _GCH // _LANES

_LN2 = 0.6931471805599453
_LOGP = (
    0.010119082927824848,
    -0.052624851367851076,
    0.13076503250423846,
    -0.2228362583280196,
    0.32697310001386687,
    -0.4992065685478449,
    0.9999574870750662,
    5.621959008883515e-07,
)


def _log16(a):
    bits = lax.bitcast_convert_type(a, jnp.int32)
    e = ((bits >> 23) - 127).astype(jnp.float32)
    m = lax.bitcast_convert_type(
        (bits & 0x007FFFFF) | 0x3F800000, jnp.float32)
    t = m - 1.0
    p = jnp.float32(_LOGP[0])
    for c in _LOGP[1:]:
        p = p * t + jnp.float32(c)
    return e * jnp.float32(_LN2) + p


def _pack16(xv, v):
    sl = pl.ds(v * _LANES, _LANES)
    accs = [None] * 4
    for i in range(_L):
        z = xv[i, sl] << (_L - 1 - i)
        k = i & 3
        accs[k] = z if accs[k] is None else accs[k] + z
    acc = (accs[0] + accs[1]) + (accs[2] + accs[3])
    return (acc + ((1 << _L) - 1)) >> 1


def _ewf_body(xt_hbm, aux_hbm, out_hbm, xv, idxv, av, outv, sem, gsem):
    wid = lax.axis_index("s") * _NC + lax.axis_index("c")
    base = pl.multiple_of(wid * _BPW, _BPW)

    cps = [
        pltpu.async_copy(
            xt_hbm.at[pl.ds(i * _BATCH + base, _BPW)], xv.at[i], sem)
        for i in range(_L)
    ]
    for cp in cps:
        cp.wait()

    gcps = []
    for g in range(_NG):
        for v8 in range(_VPG):
            idxv[g, pl.ds(v8 * _LANES, _LANES)] = _pack16(xv, g * _VPG + v8)
        gcps.append(
            pltpu.async_copy(aux_hbm.at[idxv.at[g]], av.at[g], gsem.at[g]))

    for g in range(_NG):
        gcps[g].wait()
        for v8 in range(_VPG):
            sl = pl.ds(v8 * _LANES, _LANES)
            outv[pl.ds(g * _GCH + v8 * _LANES, _LANES)] = _log16(av[g, sl])

    pltpu.sync_copy(outv, out_hbm.at[pl.ds(base, _BPW)])


@jax.jit
def _ewf_sc(xt_flat, aux):
    mesh = plsc.VectorSubcoreMesh(core_axis_name="c", subcore_axis_name="s")
    return pl.kernel(
        _ewf_body,
        out_type=jax.ShapeDtypeStruct((_BATCH,), jnp.float32),
        mesh=mesh,
        scratch_types=[
            pltpu.VMEM((_L, _BPW), jnp.int32),
            pltpu.VMEM((_NG, _GCH), jnp.int32),
            pltpu.VMEM((_NG, _GCH), jnp.float32),
            pltpu.VMEM((_BPW,), jnp.float32),
            pltpu.SemaphoreType.DMA,
            pltpu.SemaphoreType.DMA((_NG,)),
        ],
    )(xt_flat, aux)


def kernel(x, aux):
    xt = jnp.transpose(x).reshape(-1)
    return _ewf_sc(xt, aux)
